# baseline (speedup 1.0000x reference)
import functools

import jax
import jax.numpy as jnp
from jax import lax
from jax.experimental import pallas as pl
from jax.experimental.pallas import tpu as pltpu
from jax.experimental.pallas import tpu_sc as plsc

NC, NS = 2, 16
NW = NC * NS
L = 128
B, NEG, D = 4096, 200, 64
BT = B // L
N8 = NEG // 8

_mesh = plsc.VectorSubcoreMesh(
    core_axis_name="c", subcore_axis_name="s", num_cores=NC, num_subcores=NS)

@functools.partial(
    pl.kernel,
    out_type=(
        jax.ShapeDtypeStruct((B, D), jnp.float32),
        jax.ShapeDtypeStruct((B, D), jnp.float32),
        jax.ShapeDtypeStruct((NEG, 8, BT, 8, L), jnp.float32),
    ),
    mesh=_mesh,
    scratch_types=[
        pltpu.VMEM((L,), jnp.int32),
        pltpu.VMEM((N8, 8, L), jnp.int32),
        pltpu.VMEM((L, D), jnp.float32),
        pltpu.VMEM((L, D), jnp.float32),
        pltpu.VMEM((8, 8, L), jnp.float32),
        pltpu.VMEM((8, 8, L), jnp.float32),
        pltpu.SemaphoreType.DMA,
        pltpu.SemaphoreType.DMA,
        pltpu.SemaphoreType.DMA,
        pltpu.SemaphoreType.DMA,
    ],
    compiler_params=pltpu.CompilerParams(
        use_tc_tiling_on_sc=False, needs_layout_passes=False),
)
def _gather_kernel(head_idx, rel_idx, neg_q, ent, rel,
                   head_out, rel_out, q_out,
                   hidx_v, idx_v, rows0, rows1, blk0, blk1,
                   gsem0, gsem1, wsem0, wsem1):
    wid = lax.axis_index("s") * NC + lax.axis_index("c")
    rows = (rows0, rows1)
    blks = (blk0, blk1)
    gsems = (gsem0, gsem1)
    wsems = (wsem0, wsem1)

    pltpu.sync_copy(head_idx.at[wid], hidx_v)
    pltpu.async_copy(ent.at[hidx_v], rows0, gsem0).wait()
    pltpu.async_copy(rows0, head_out.at[pl.ds(wid * L, L)], wsem0).wait()

    pltpu.sync_copy(rel_idx.at[wid], hidx_v)
    pltpu.async_copy(rel.at[hidx_v], rows0, gsem0).wait()
    pltpu.async_copy(rows0, rel_out.at[pl.ds(wid * L, L)], wsem0).wait()

    pltpu.sync_copy(neg_q.at[:, wid], idx_v)

    def fire_gather(n, h):
        pltpu.async_copy(ent.at[idx_v.at[n // 8, n % 8]], rows[h], gsems[h])

    def drain_gather(h):
        pltpu.make_async_copy(ent.at[hidx_v], rows[h], gsems[h]).wait()

    iota16 = jnp.arange(16, dtype=jnp.int32)

    def transpose(h):
        src, dst = rows[h], blks[h]
        for f in range(D):
            for m in range(L // 16):
                vals = plsc.load_gather(
                    src, [iota16 + (16 * m), jnp.full((16,), f, jnp.int32)])
                dst[f // 8, f % 8, pl.ds(16 * m, 16)] = vals

    def fire_writeback(n, h):
        pltpu.async_copy(blks[h], q_out.at[n, :, wid], wsems[h])

    def drain_writeback(n, h):
        pltpu.make_async_copy(blks[h], q_out.at[n, :, wid], wsems[h]).wait()

    fire_gather(0, 0)

    def body(p, carry):
        for h in (0, 1):
            n = 2 * p + h

            @pl.when(n + 1 < NEG)
            def _():
                fire_gather(n + 1, 1 - h)

            drain_gather(h)

            @pl.when(n >= 2)
            def _():
                drain_writeback(n - 2, h)

            transpose(h)
            fire_writeback(n, h)
        return carry

    lax.fori_loop(0, NEG // 2, body, 0)
    drain_writeback(NEG - 2, 0)
    drain_writeback(NEG - 1, 1)


def kernel(sample, negative_sample, entity_embedding, relation_embedding):
    head_idx = sample[:, 0].astype(jnp.int32).reshape(BT, L)
    rel_idx = sample[:, 1].astype(jnp.int32).reshape(BT, L)
    neg_q = (negative_sample.astype(jnp.int32).T
             .reshape(N8, 8, BT, L).transpose(0, 2, 1, 3))
    head, relation, q = _gather_kernel(
        head_idx, rel_idx, neg_q, entity_embedding, relation_embedding)
    tail = q.transpose(2, 4, 0, 1, 3).reshape(B, NEG, D)
    return head[:, None, :], relation[:, None, :], tail
